# baseline (speedup 1.0000x reference)
import functools

import jax
import jax.numpy as jnp
from jax import lax
from jax.experimental import pallas as pl
from jax.experimental.pallas import tpu as pltpu
from jax.experimental.pallas import tpu_sc as plsc

_BATCH = 4096
_SEQ = 200
_DIM = 64
_B = _BATCH * _SEQ
_NC, _NS = 2, 16
_NW = _NC * _NS
_ROWS_PER_W = _B // _NW
_G = 128
_K = _ROWS_PER_W // _G


def _gather_kernel(idx_hbm, table_hbm, out_hbm, idx_v, rows_v, gsem):
    wid = lax.axis_index("s") * _NC + lax.axis_index("c")
    base = wid * _ROWS_PER_W
    pltpu.sync_copy(idx_hbm.at[wid], idx_v)

    def body(j, carry):
        pltpu.async_copy(table_hbm.at[idx_v.at[j]], rows_v, gsem).wait()
        pltpu.sync_copy(rows_v, out_hbm.at[pl.ds(base + j * _G, _G)])
        return carry

    lax.fori_loop(0, _K, body, 0)


def kernel(token_ids, embedding_table):
    idx = token_ids.reshape(_NW, _K, _G).astype(jnp.int32)
    mesh = plsc.VectorSubcoreMesh(core_axis_name="c", subcore_axis_name="s")
    run = functools.partial(
        pl.kernel,
        mesh=mesh,
        out_type=jax.ShapeDtypeStruct((_B, _DIM), jnp.float32),
        scratch_types=[
            pltpu.VMEM((_K, _G), jnp.int32),
            pltpu.VMEM((_G, _DIM), jnp.float32),
            pltpu.SemaphoreType.DMA,
        ],
        compiler_params=pltpu.CompilerParams(use_tc_tiling_on_sc=False),
    )(_gather_kernel)
    out = run(idx, embedding_table)
    return out.reshape(_BATCH, _SEQ, _DIM)


# speedup vs baseline: 1.1079x; 1.1079x over previous
import functools

import jax
import jax.numpy as jnp
from jax import lax
from jax.experimental import pallas as pl
from jax.experimental.pallas import tpu as pltpu
from jax.experimental.pallas import tpu_sc as plsc

_BATCH = 4096
_SEQ = 200
_DIM = 64
_B = _BATCH * _SEQ
_NC, _NS = 2, 16
_NW = _NC * _NS
_ROWS_PER_W = _B // _NW
_G = 128
_CPG = 4
_GROUP = _G * _CPG
_NGRP = _ROWS_PER_W // _GROUP
_NPAIR = _NGRP // 2 - 1


def _gather_kernel(idx_hbm, table_hbm, out_hbm,
                   idx_v, buf0, buf1, gsem0, gsem1, ssem0, ssem1):
    wid = lax.axis_index("s") * _NC + lax.axis_index("c")
    base = wid * _ROWS_PER_W
    pltpu.sync_copy(idx_hbm.at[wid], idx_v)

    def fire_gathers(g, buf, sem):
        for c in range(_CPG):
            pltpu.async_copy(
                table_hbm.at[idx_v.at[g * _CPG + c]],
                buf.at[pl.ds(c * _G, _G)], sem)

    def wait_gathers(buf, sem):
        pltpu.make_async_copy(table_hbm.at[pl.ds(0, _GROUP)], buf, sem).wait()

    def fire_store(g, buf, sem):
        pltpu.async_copy(buf, out_hbm.at[pl.ds(base + g * _GROUP, _GROUP)], sem)

    def wait_store(buf, sem):
        pltpu.make_async_copy(buf, out_hbm.at[pl.ds(base, _GROUP)], sem).wait()

    fire_gathers(0, buf0, gsem0)
    fire_gathers(1, buf1, gsem1)

    def body(p, carry):
        g = 2 * p
        wait_gathers(buf0, gsem0)
        fire_store(g, buf0, ssem0)
        wait_gathers(buf1, gsem1)
        fire_store(g + 1, buf1, ssem1)
        wait_store(buf0, ssem0)
        fire_gathers(g + 2, buf0, gsem0)
        wait_store(buf1, ssem1)
        fire_gathers(g + 3, buf1, gsem1)
        return carry

    lax.fori_loop(0, _NPAIR, body, 0)

    wait_gathers(buf0, gsem0)
    fire_store(_NGRP - 2, buf0, ssem0)
    wait_gathers(buf1, gsem1)
    fire_store(_NGRP - 1, buf1, ssem1)
    wait_store(buf0, ssem0)
    wait_store(buf1, ssem1)


def kernel(token_ids, embedding_table):
    idx = token_ids.reshape(_NW, _ROWS_PER_W // _G, _G).astype(jnp.int32)
    mesh = plsc.VectorSubcoreMesh(core_axis_name="c", subcore_axis_name="s")
    run = functools.partial(
        pl.kernel,
        mesh=mesh,
        out_type=jax.ShapeDtypeStruct((_B, _DIM), jnp.float32),
        scratch_types=[
            pltpu.VMEM((_ROWS_PER_W // _G, _G), jnp.int32),
            pltpu.VMEM((_GROUP, _DIM), jnp.float32),
            pltpu.VMEM((_GROUP, _DIM), jnp.float32),
            pltpu.SemaphoreType.DMA,
            pltpu.SemaphoreType.DMA,
            pltpu.SemaphoreType.DMA,
            pltpu.SemaphoreType.DMA,
        ],
        compiler_params=pltpu.CompilerParams(use_tc_tiling_on_sc=False),
    )(_gather_kernel)
    out = run(idx, embedding_table)
    return out.reshape(_BATCH, _SEQ, _DIM)
